# baseline (speedup 1.0000x reference)
import dataclasses
import functools

import jax
import jax.numpy as jnp
from jax.experimental import pallas as pl
from jax.experimental.pallas import tpu as pltpu
from jax.experimental.pallas import tpu_sc as plsc

IN_F = 4096
OUT_F = 1024
BATCH = 16384
ROWS_PER_STEP = 8
LANES = 16


def _argmax_body(w_ref, o_ref):
    w = w_ref[...]
    m = jnp.max(w, axis=1, keepdims=True)
    ii = jax.lax.broadcasted_iota(jnp.int32, w.shape, 1)
    cand = jnp.where(w == m, ii, IN_F)
    o_ref[0, 0, :] = jnp.min(cand, axis=1).astype(jnp.int32)


def _argmax(raw_weight):
    out = pl.pallas_call(
        _argmax_body,
        grid=(OUT_F // 128,),
        in_specs=[pl.BlockSpec((128, IN_F), lambda i: (i, 0))],
        out_specs=pl.BlockSpec((1, 1, 128), lambda i: (i, 0, 0)),
        out_shape=jax.ShapeDtypeStruct((OUT_F // 128, 1, 128), jnp.int32),
    )(raw_weight)
    return out.reshape(1, OUT_F)


def _gather_sc(x, idx):
    mesh = plsc.VectorSubcoreMesh(core_axis_name="c", subcore_axis_name="s")
    cp = pltpu.CompilerParams()
    if "needs_layout_passes" in pltpu.CompilerParams.__dataclass_fields__:
        cp = dataclasses.replace(cp, needs_layout_passes=False)

    @functools.partial(
        pl.kernel,
        mesh=mesh,
        out_type=jax.ShapeDtypeStruct((BATCH, OUT_F), jnp.float32),
        scratch_types=[
            pltpu.VMEM((1, OUT_F), jnp.int32),
            pltpu.SemaphoreType.DMA,
        ],
        compiler_params=cp,
    )
    def k(i_hbm, x_hbm, o_hbm, i_vmem, sem):
        pltpu.async_copy(i_hbm, i_vmem, sem).wait()
        idx_ref = i_vmem.at[0]

        def body(x_vmem, o_vmem):
            for j in range(OUT_F // LANES):
                cols = idx_ref[pl.ds(j * LANES, LANES)]
                for r in range(ROWS_PER_STEP):
                    rows = jnp.full((LANES,), r, jnp.int32)
                    vals = plsc.load_gather(x_vmem, [rows, cols])
                    o_vmem[r, pl.ds(j * LANES, LANES)] = vals

        pltpu.emit_pipeline(
            body,
            grid=(BATCH // ROWS_PER_STEP,),
            in_specs=[
                pl.BlockSpec((ROWS_PER_STEP, IN_F), lambda i: (i, 0)),
            ],
            out_specs=[pl.BlockSpec((ROWS_PER_STEP, OUT_F), lambda i: (i, 0))],
            core_axis_name=("c", "s"),
            dimension_semantics=(pltpu.PARALLEL,),
        )(x_hbm, o_hbm)

    return k(idx, x)


def kernel(x, raw_weight):
    idx = _argmax(raw_weight)
    return _gather_sc(x, idx)


# speedup vs baseline: 1.2470x; 1.2470x over previous
import dataclasses
import functools

import jax
import jax.numpy as jnp
from jax.experimental import pallas as pl
from jax.experimental.pallas import tpu as pltpu
from jax.experimental.pallas import tpu_sc as plsc

IN_F = 4096
OUT_F = 1024
BATCH = 16384
ROWS_PER_STEP = 8
LANES = 16


def _argmax_body(w_ref, o_ref):
    w = w_ref[...]
    m = jnp.max(w, axis=1, keepdims=True)
    ii = jax.lax.broadcasted_iota(jnp.int32, w.shape, 1)
    cand = jnp.where(w == m, ii, IN_F)
    o_ref[0, 0, :] = jnp.min(cand, axis=1).astype(jnp.int32)


def _argmax(raw_weight):
    out = pl.pallas_call(
        _argmax_body,
        grid=(OUT_F // 128,),
        in_specs=[pl.BlockSpec((128, IN_F), lambda i: (i, 0))],
        out_specs=pl.BlockSpec((1, 1, 128), lambda i: (i, 0, 0)),
        out_shape=jax.ShapeDtypeStruct((OUT_F // 128, 1, 128), jnp.int32),
    )(raw_weight)
    return out.reshape(1, OUT_F)


def _gather_sc(x, idx):
    mesh = plsc.VectorSubcoreMesh(core_axis_name="c", subcore_axis_name="s")
    cp = pltpu.CompilerParams()
    if "needs_layout_passes" in pltpu.CompilerParams.__dataclass_fields__:
        cp = dataclasses.replace(cp, needs_layout_passes=False)

    @functools.partial(
        pl.kernel,
        mesh=mesh,
        out_type=jax.ShapeDtypeStruct((BATCH, OUT_F), jnp.float32),
        scratch_types=[
            pltpu.VMEM((1, OUT_F), jnp.int32),
            pltpu.SemaphoreType.DMA,
        ],
        compiler_params=cp,
    )
    def k(i_hbm, x_hbm, o_hbm, i_vmem, sem):
        pltpu.async_copy(i_hbm, i_vmem, sem).wait()
        idx_ref = i_vmem.at[0]

        def body(x_vmem, o_vmem):
            @pl.loop(0, OUT_F // LANES)
            def _(j):
                cols = idx_ref[pl.ds(j * LANES, LANES)]
                for r in range(ROWS_PER_STEP):
                    rows = jnp.full((LANES,), r, jnp.int32)
                    vals = plsc.load_gather(x_vmem, [rows, cols])
                    o_vmem[r, pl.ds(j * LANES, LANES)] = vals

        pltpu.emit_pipeline(
            body,
            grid=(BATCH // ROWS_PER_STEP,),
            in_specs=[
                pl.BlockSpec((ROWS_PER_STEP, IN_F), lambda i: (i, 0)),
            ],
            out_specs=[pl.BlockSpec((ROWS_PER_STEP, OUT_F), lambda i: (i, 0))],
            core_axis_name=("c", "s"),
            dimension_semantics=(pltpu.PARALLEL,),
        )(x_hbm, o_hbm)

    return k(idx, x)


def kernel(x, raw_weight):
    idx = _argmax(raw_weight)
    return _gather_sc(x, idx)


# speedup vs baseline: 1.2516x; 1.0037x over previous
import dataclasses
import functools

import jax
import jax.numpy as jnp
from jax.experimental import pallas as pl
from jax.experimental.pallas import tpu as pltpu
from jax.experimental.pallas import tpu_sc as plsc

IN_F = 4096
OUT_F = 1024
BATCH = 16384
ROWS_PER_STEP = 8
LANES = 16


def _argmax_body(w_ref, o_ref):
    w = w_ref[...]
    m = jnp.max(w, axis=1, keepdims=True)
    ii = jax.lax.broadcasted_iota(jnp.int32, w.shape, 1)
    cand = jnp.where(w == m, ii, IN_F)
    o_ref[0, 0, :] = jnp.min(cand, axis=1).astype(jnp.int32)


def _argmax(raw_weight):
    out = pl.pallas_call(
        _argmax_body,
        grid=(OUT_F // 128,),
        in_specs=[pl.BlockSpec((128, IN_F), lambda i: (i, 0))],
        out_specs=pl.BlockSpec((1, 1, 128), lambda i: (i, 0, 0)),
        out_shape=jax.ShapeDtypeStruct((OUT_F // 128, 1, 128), jnp.int32),
    )(raw_weight)
    return out.reshape(1, OUT_F)


def _gather_sc(x, idx):
    mesh = plsc.VectorSubcoreMesh(core_axis_name="c", subcore_axis_name="s")
    cp = pltpu.CompilerParams()
    if "needs_layout_passes" in pltpu.CompilerParams.__dataclass_fields__:
        cp = dataclasses.replace(cp, needs_layout_passes=False)

    n_tiles = 32
    rows_per_tile = BATCH // n_tiles
    n_chunks = rows_per_tile // ROWS_PER_STEP

    @functools.partial(
        pl.kernel,
        mesh=mesh,
        out_type=jax.ShapeDtypeStruct((BATCH, OUT_F), jnp.float32),
        scratch_types=[
            pltpu.VMEM((1, OUT_F), jnp.int32),
            pltpu.VMEM((2, ROWS_PER_STEP, IN_F), jnp.float32),
            pltpu.VMEM((2, ROWS_PER_STEP, OUT_F), jnp.float32),
            pltpu.SemaphoreType.DMA,
            pltpu.SemaphoreType.DMA,
            pltpu.SemaphoreType.DMA,
            pltpu.SemaphoreType.DMA,
        ],
        compiler_params=cp,
    )
    def k(i_hbm, x_hbm, o_hbm, i_vmem, xb, ob, sin0, sin1, sout0, sout1):
        pltpu.async_copy(i_hbm, i_vmem, sin0).wait()
        idx_ref = i_vmem.at[0]
        sin = (sin0, sin1)
        sout = (sout0, sout1)

        wid = jax.lax.axis_index("s") * 2 + jax.lax.axis_index("c")
        base = wid * rows_per_tile

        def in_copy(chunk, buf):
            return pltpu.make_async_copy(
                x_hbm.at[pl.ds(base + chunk * ROWS_PER_STEP, ROWS_PER_STEP)],
                xb.at[buf], sin[buf])

        def out_copy(chunk, buf):
            return pltpu.make_async_copy(
                ob.at[buf],
                o_hbm.at[pl.ds(base + chunk * ROWS_PER_STEP, ROWS_PER_STEP)],
                sout[buf])

        in_copy(0, 0).start()

        @pl.loop(0, n_chunks, step=2)
        def _(g):
            for b in range(2):
                gi = g + b

                @pl.when(gi + 1 < n_chunks)
                def _():
                    in_copy(gi + 1, 1 - b).start()

                in_copy(gi, b).wait()

                @pl.when(gi >= 2)
                def _():
                    out_copy(gi - 2, b).wait()

                x_vmem = xb.at[b]
                o_vmem = ob.at[b]

                @pl.loop(0, OUT_F // LANES)
                def _(j):
                    cols = idx_ref[pl.ds(j * LANES, LANES)]
                    for r in range(ROWS_PER_STEP):
                        rows = jnp.full((LANES,), r, jnp.int32)
                        vals = plsc.load_gather(x_vmem, [rows, cols])
                        o_vmem[r, pl.ds(j * LANES, LANES)] = vals

                out_copy(gi, b).start()

        out_copy(n_chunks - 2, 0).wait()
        out_copy(n_chunks - 1, 1).wait()

    return k(idx, x)


def kernel(x, raw_weight):
    idx = _argmax(raw_weight)
    return _gather_sc(x, idx)


# speedup vs baseline: 1.6940x; 1.3535x over previous
import dataclasses
import functools

import jax
import jax.numpy as jnp
from jax.experimental import pallas as pl
from jax.experimental.pallas import tpu as pltpu
from jax.experimental.pallas import tpu_sc as plsc

IN_F = 4096
OUT_F = 1024
BATCH = 16384
ROWS_PER_STEP = 8
LANES = 16


def _argmax_body(w_ref, o_ref):
    w = w_ref[...]
    m = jnp.max(w, axis=1, keepdims=True)
    ii = jax.lax.broadcasted_iota(jnp.int32, w.shape, 1)
    cand = jnp.where(w == m, ii, IN_F)
    o_ref[0, 0, :] = jnp.min(cand, axis=1).astype(jnp.int32)


def _argmax(raw_weight):
    out = pl.pallas_call(
        _argmax_body,
        grid=(OUT_F // 128,),
        in_specs=[pl.BlockSpec((128, IN_F), lambda i: (i, 0))],
        out_specs=pl.BlockSpec((1, 1, 128), lambda i: (i, 0, 0)),
        out_shape=jax.ShapeDtypeStruct((OUT_F // 128, 1, 128), jnp.int32),
    )(raw_weight)
    return out.reshape(1, OUT_F)


def _gather_sc(x, idx):
    mesh = plsc.VectorSubcoreMesh(core_axis_name="c", subcore_axis_name="s")
    cp = pltpu.CompilerParams()
    if "needs_layout_passes" in pltpu.CompilerParams.__dataclass_fields__:
        cp = dataclasses.replace(cp, needs_layout_passes=False)

    n_tiles = 32
    rows_per_tile = BATCH // n_tiles
    n_chunks = rows_per_tile // ROWS_PER_STEP

    @functools.partial(
        pl.kernel,
        mesh=mesh,
        out_type=jax.ShapeDtypeStruct((BATCH, OUT_F), jnp.float32),
        scratch_types=[
            pltpu.VMEM((1, OUT_F), jnp.int32),
            pltpu.VMEM((2, ROWS_PER_STEP, IN_F), jnp.float32),
            pltpu.VMEM((2, ROWS_PER_STEP, OUT_F), jnp.float32),
            pltpu.SemaphoreType.DMA,
            pltpu.SemaphoreType.DMA,
            pltpu.SemaphoreType.DMA,
            pltpu.SemaphoreType.DMA,
        ],
        compiler_params=cp,
    )
    def k(i_hbm, x_hbm, o_hbm, i_vmem, xb, ob, sin0, sin1, sout0, sout1):
        pltpu.async_copy(i_hbm, i_vmem, sin0).wait()
        idx_ref = i_vmem.at[0]
        sin = (sin0, sin1)
        sout = (sout0, sout1)

        wid = jax.lax.axis_index("s") * 2 + jax.lax.axis_index("c")
        base = wid * rows_per_tile

        def in_copy(chunk, buf):
            return pltpu.make_async_copy(
                x_hbm.at[pl.ds(base + chunk * ROWS_PER_STEP, ROWS_PER_STEP)],
                xb.at[buf], sin[buf])

        def out_copy(chunk, buf):
            return pltpu.make_async_copy(
                ob.at[buf],
                o_hbm.at[pl.ds(base + chunk * ROWS_PER_STEP, ROWS_PER_STEP)],
                sout[buf])

        in_copy(0, 0).start()

        @pl.loop(0, n_chunks, step=2)
        def _(g):
            for b in range(2):
                gi = g + b

                @pl.when(gi + 1 < n_chunks)
                def _():
                    in_copy(gi + 1, 1 - b).start()

                in_copy(gi, b).wait()

                @pl.when(gi >= 2)
                def _():
                    out_copy(gi - 2, b).wait()

                x_vmem = xb.at[b]
                o_vmem = ob.at[b]

                @plsc.parallel_loop(0, OUT_F // LANES, unroll=4)
                def _(j):
                    cols = idx_ref[pl.ds(j * LANES, LANES)]
                    for r in range(ROWS_PER_STEP):
                        rows = jnp.full((LANES,), r, jnp.int32)
                        vals = plsc.load_gather(x_vmem, [rows, cols])
                        o_vmem[r, pl.ds(j * LANES, LANES)] = vals

                out_copy(gi, b).start()

        out_copy(n_chunks - 2, 0).wait()
        out_copy(n_chunks - 1, 1).wait()

    return k(idx, x)


def kernel(x, raw_weight):
    idx = _argmax(raw_weight)
    return _gather_sc(x, idx)


# speedup vs baseline: 1.7003x; 1.0037x over previous
import dataclasses
import functools

import jax
import jax.numpy as jnp
from jax.experimental import pallas as pl
from jax.experimental.pallas import tpu as pltpu
from jax.experimental.pallas import tpu_sc as plsc

IN_F = 4096
OUT_F = 1024
BATCH = 16384
ROWS_PER_STEP = 8
LANES = 16


def _argmax_body(w_ref, o_ref):
    w = w_ref[...]
    m = jnp.max(w, axis=1, keepdims=True)
    ii = jax.lax.broadcasted_iota(jnp.int32, w.shape, 1)
    cand = jnp.where(w == m, ii, IN_F)
    o_ref[0, 0, :] = jnp.min(cand, axis=1).astype(jnp.int32)


def _argmax(raw_weight):
    out = pl.pallas_call(
        _argmax_body,
        grid=(OUT_F // 128,),
        in_specs=[pl.BlockSpec((128, IN_F), lambda i: (i, 0))],
        out_specs=pl.BlockSpec((1, 1, 128), lambda i: (i, 0, 0)),
        out_shape=jax.ShapeDtypeStruct((OUT_F // 128, 1, 128), jnp.int32),
    )(raw_weight)
    return out.reshape(1, OUT_F)


def _gather_sc(x, idx):
    mesh = plsc.VectorSubcoreMesh(core_axis_name="c", subcore_axis_name="s")
    cp = pltpu.CompilerParams()
    if "needs_layout_passes" in pltpu.CompilerParams.__dataclass_fields__:
        cp = dataclasses.replace(cp, needs_layout_passes=False)

    n_tiles = 32
    rows_per_tile = BATCH // n_tiles
    n_chunks = rows_per_tile // ROWS_PER_STEP

    @functools.partial(
        pl.kernel,
        mesh=mesh,
        out_type=jax.ShapeDtypeStruct((BATCH, OUT_F), jnp.float32),
        scratch_types=[
            pltpu.VMEM((1, OUT_F), jnp.int32),
            pltpu.VMEM((2, ROWS_PER_STEP, IN_F), jnp.float32),
            pltpu.VMEM((2, ROWS_PER_STEP, OUT_F), jnp.float32),
            pltpu.SemaphoreType.DMA,
            pltpu.SemaphoreType.DMA,
            pltpu.SemaphoreType.DMA,
            pltpu.SemaphoreType.DMA,
        ],
        compiler_params=cp,
    )
    def k(i_hbm, x_hbm, o_hbm, i_vmem, xb, ob, sin0, sin1, sout0, sout1):
        pltpu.async_copy(i_hbm, i_vmem, sin0).wait()
        idx_ref = i_vmem.at[0]
        sin = (sin0, sin1)
        sout = (sout0, sout1)

        wid = jax.lax.axis_index("s") * 2 + jax.lax.axis_index("c")
        base = wid * rows_per_tile

        def in_copy(chunk, buf):
            return pltpu.make_async_copy(
                x_hbm.at[pl.ds(base + chunk * ROWS_PER_STEP, ROWS_PER_STEP)],
                xb.at[buf], sin[buf])

        def out_copy(chunk, buf):
            return pltpu.make_async_copy(
                ob.at[buf],
                o_hbm.at[pl.ds(base + chunk * ROWS_PER_STEP, ROWS_PER_STEP)],
                sout[buf])

        in_copy(0, 0).start()

        @pl.loop(0, n_chunks, step=2)
        def _(g):
            for b in range(2):
                gi = g + b

                @pl.when(gi + 1 < n_chunks)
                def _():
                    in_copy(gi + 1, 1 - b).start()

                in_copy(gi, b).wait()

                @pl.when(gi >= 2)
                def _():
                    out_copy(gi - 2, b).wait()

                x_vmem = xb.at[b]
                o_vmem = ob.at[b]

                @plsc.parallel_loop(0, OUT_F // LANES, unroll=8)
                def _(j):
                    cols = idx_ref[pl.ds(j * LANES, LANES)]
                    for r in range(ROWS_PER_STEP):
                        rows = jnp.full((LANES,), r, jnp.int32)
                        vals = plsc.load_gather(x_vmem, [rows, cols])
                        o_vmem[r, pl.ds(j * LANES, LANES)] = vals

                out_copy(gi, b).start()

        out_copy(n_chunks - 2, 0).wait()
        out_copy(n_chunks - 1, 1).wait()

    return k(idx, x)


def kernel(x, raw_weight):
    idx = _argmax(raw_weight)
    return _gather_sc(x, idx)


# speedup vs baseline: 1.7073x; 1.0042x over previous
import dataclasses
import functools

import jax
import jax.numpy as jnp
from jax.experimental import pallas as pl
from jax.experimental.pallas import tpu as pltpu
from jax.experimental.pallas import tpu_sc as plsc

IN_F = 4096
OUT_F = 1024
BATCH = 16384
ROWS_PER_STEP = 8
LANES = 16


def _argmax_body(w_ref, o_ref):
    w = w_ref[...]
    m = jnp.max(w, axis=1, keepdims=True)
    ii = jax.lax.broadcasted_iota(jnp.int32, w.shape, 1)
    cand = jnp.where(w == m, ii, IN_F)
    am = jnp.min(cand, axis=1).astype(jnp.int32)
    o_ref[...] = am.reshape(OUT_F // 128, 128)


def _argmax(raw_weight):
    return pl.pallas_call(
        _argmax_body,
        out_shape=jax.ShapeDtypeStruct((OUT_F // 128, 128), jnp.int32),
    )(raw_weight)


def _gather_sc(x, idx):
    mesh = plsc.VectorSubcoreMesh(core_axis_name="c", subcore_axis_name="s")
    cp = pltpu.CompilerParams()
    if "needs_layout_passes" in pltpu.CompilerParams.__dataclass_fields__:
        cp = dataclasses.replace(cp, needs_layout_passes=False)

    n_tiles = 32
    rows_per_tile = BATCH // n_tiles
    n_chunks = rows_per_tile // ROWS_PER_STEP

    @functools.partial(
        pl.kernel,
        mesh=mesh,
        out_type=jax.ShapeDtypeStruct((BATCH, OUT_F), jnp.float32),
        scratch_types=[
            pltpu.VMEM((OUT_F // 128, 128), jnp.int32),
            pltpu.VMEM((1, OUT_F), jnp.int32),
            pltpu.VMEM((2, ROWS_PER_STEP, IN_F), jnp.float32),
            pltpu.VMEM((2, ROWS_PER_STEP, OUT_F), jnp.float32),
            pltpu.SemaphoreType.DMA,
            pltpu.SemaphoreType.DMA,
            pltpu.SemaphoreType.DMA,
            pltpu.SemaphoreType.DMA,
        ],
        compiler_params=cp,
    )
    def k(i_hbm, x_hbm, o_hbm, i8_vmem, i_vmem, xb, ob, sin0, sin1, sout0,
          sout1):
        pltpu.async_copy(i_hbm, i8_vmem, sin0).wait()
        for rr in range(OUT_F // 128):
            for kk in range(128 // LANES):
                i_vmem[0, pl.ds(rr * 128 + kk * LANES, LANES)] = (
                    i8_vmem[rr, pl.ds(kk * LANES, LANES)])
        idx_ref = i_vmem.at[0]
        sin = (sin0, sin1)
        sout = (sout0, sout1)

        wid = jax.lax.axis_index("s") * 2 + jax.lax.axis_index("c")
        base = wid * rows_per_tile

        def in_copy(chunk, buf):
            return pltpu.make_async_copy(
                x_hbm.at[pl.ds(base + chunk * ROWS_PER_STEP, ROWS_PER_STEP)],
                xb.at[buf], sin[buf])

        def out_copy(chunk, buf):
            return pltpu.make_async_copy(
                ob.at[buf],
                o_hbm.at[pl.ds(base + chunk * ROWS_PER_STEP, ROWS_PER_STEP)],
                sout[buf])

        in_copy(0, 0).start()

        @pl.loop(0, n_chunks, step=2)
        def _(g):
            for b in range(2):
                gi = g + b

                @pl.when(gi + 1 < n_chunks)
                def _():
                    in_copy(gi + 1, 1 - b).start()

                in_copy(gi, b).wait()

                @pl.when(gi >= 2)
                def _():
                    out_copy(gi - 2, b).wait()

                x_vmem = xb.at[b]
                o_vmem = ob.at[b]

                @plsc.parallel_loop(0, OUT_F // LANES, unroll=8)
                def _(j):
                    cols = idx_ref[pl.ds(j * LANES, LANES)]
                    for r in range(ROWS_PER_STEP):
                        rows = jnp.full((LANES,), r, jnp.int32)
                        vals = plsc.load_gather(x_vmem, [rows, cols])
                        o_vmem[r, pl.ds(j * LANES, LANES)] = vals

                out_copy(gi, b).start()

        out_copy(n_chunks - 2, 0).wait()
        out_copy(n_chunks - 1, 1).wait()

    return k(idx, x)


def kernel(x, raw_weight):
    idx = _argmax(raw_weight)
    return _gather_sc(x, idx)


# speedup vs baseline: 1.7662x; 1.0345x over previous
import dataclasses
import functools

import jax
import jax.numpy as jnp
from jax.experimental import pallas as pl
from jax.experimental.pallas import tpu as pltpu
from jax.experimental.pallas import tpu_sc as plsc

IN_F = 4096
OUT_F = 1024
BATCH = 16384
ROWS_PER_STEP = 8
LANES = 16


def _argmax_body(w_ref, o_ref):
    w = w_ref[...]
    m = jnp.max(w, axis=1, keepdims=True)
    ii = jax.lax.broadcasted_iota(jnp.int32, w.shape, 1)
    cand = jnp.where(w == m, ii, IN_F)
    am = jnp.min(cand, axis=1).astype(jnp.int32)
    o_ref[...] = am.reshape(OUT_F // 128, 128)


def _argmax(raw_weight):
    return pl.pallas_call(
        _argmax_body,
        out_shape=jax.ShapeDtypeStruct((OUT_F // 128, 128), jnp.int32),
    )(raw_weight)


def _gather_sc(x, idx):
    mesh = plsc.VectorSubcoreMesh(core_axis_name="c", subcore_axis_name="s")
    cp = pltpu.CompilerParams()
    if "needs_layout_passes" in pltpu.CompilerParams.__dataclass_fields__:
        cp = dataclasses.replace(cp, needs_layout_passes=False)

    n_tiles = 32
    rows_per_tile = BATCH // n_tiles
    n_chunks = rows_per_tile // ROWS_PER_STEP

    @functools.partial(
        pl.kernel,
        mesh=mesh,
        out_type=jax.ShapeDtypeStruct((BATCH, OUT_F), jnp.float32),
        scratch_types=[
            pltpu.VMEM((OUT_F // 128, 128), jnp.int32),
            pltpu.VMEM((1, OUT_F), jnp.int32),
            pltpu.VMEM((2, ROWS_PER_STEP, IN_F), jnp.float32),
            pltpu.VMEM((2, ROWS_PER_STEP, OUT_F), jnp.float32),
            pltpu.SemaphoreType.DMA,
            pltpu.SemaphoreType.DMA,
            pltpu.SemaphoreType.DMA,
            pltpu.SemaphoreType.DMA,
            pltpu.SemaphoreType.DMA,
            pltpu.SemaphoreType.DMA,
        ],
        compiler_params=cp,
    )
    def k(i_hbm, x_hbm, o_hbm, i8_vmem, i_vmem, xb, ob, sin0, sin1, sin2,
          sin3, sout0, sout1):
        pltpu.async_copy(i_hbm, i8_vmem, sin0).wait()
        for rr in range(OUT_F // 128):
            for kk in range(128 // LANES):
                i_vmem[0, pl.ds(rr * 128 + kk * LANES, LANES)] = (
                    i8_vmem[rr, pl.ds(kk * LANES, LANES)])
        idx_ref = i_vmem.at[0]
        sina = (sin0, sin1)
        sinb = (sin2, sin3)
        sout = (sout0, sout1)
        half = ROWS_PER_STEP // 2

        wid = jax.lax.axis_index("s") * 2 + jax.lax.axis_index("c")
        base = wid * rows_per_tile

        def in_copies(chunk, buf):
            row0 = base + chunk * ROWS_PER_STEP
            return (
                pltpu.make_async_copy(
                    x_hbm.at[pl.ds(row0, half)],
                    xb.at[buf].at[pl.ds(0, half)], sina[buf]),
                pltpu.make_async_copy(
                    x_hbm.at[pl.ds(row0 + half, half)],
                    xb.at[buf].at[pl.ds(half, half)], sinb[buf]),
            )

        def out_copy(chunk, buf):
            return pltpu.make_async_copy(
                ob.at[buf],
                o_hbm.at[pl.ds(base + chunk * ROWS_PER_STEP, ROWS_PER_STEP)],
                sout[buf])

        for c in in_copies(0, 0):
            c.start()

        @pl.loop(0, n_chunks, step=2)
        def _(g):
            for b in range(2):
                gi = g + b

                @pl.when(gi + 1 < n_chunks)
                def _():
                    for c in in_copies(gi + 1, 1 - b):
                        c.start()

                for c in in_copies(gi, b):
                    c.wait()

                @pl.when(gi >= 2)
                def _():
                    out_copy(gi - 2, b).wait()

                x_vmem = xb.at[b]
                o_vmem = ob.at[b]

                @plsc.parallel_loop(0, OUT_F // LANES, unroll=8)
                def _(j):
                    cols = idx_ref[pl.ds(j * LANES, LANES)]
                    for r in range(ROWS_PER_STEP):
                        rows = jnp.full((LANES,), r, jnp.int32)
                        vals = plsc.load_gather(x_vmem, [rows, cols])
                        o_vmem[r, pl.ds(j * LANES, LANES)] = vals

                out_copy(gi, b).start()

        out_copy(n_chunks - 2, 0).wait()
        out_copy(n_chunks - 1, 1).wait()

    return k(idx, x)


def kernel(x, raw_weight):
    idx = _argmax(raw_weight)
    return _gather_sc(x, idx)


# speedup vs baseline: 1.8034x; 1.0211x over previous
import dataclasses
import functools

import jax
import jax.numpy as jnp
from jax.experimental import pallas as pl
from jax.experimental.pallas import tpu as pltpu
from jax.experimental.pallas import tpu_sc as plsc

IN_F = 4096
OUT_F = 1024
BATCH = 16384
ROWS_PER_STEP = 8
LANES = 16


def _argmax_body(w_ref, o_ref):
    w = w_ref[...]
    m = jnp.max(w, axis=1, keepdims=True)
    ii = jax.lax.broadcasted_iota(jnp.int32, w.shape, 1)
    cand = jnp.where(w == m, ii, IN_F)
    am = jnp.min(cand, axis=1).astype(jnp.int32)
    o_ref[...] = am.reshape(OUT_F // 128, 128)


def _argmax(raw_weight):
    return pl.pallas_call(
        _argmax_body,
        out_shape=jax.ShapeDtypeStruct((OUT_F // 128, 128), jnp.int32),
    )(raw_weight)


def _gather_sc(x, idx):
    mesh = plsc.VectorSubcoreMesh(core_axis_name="c", subcore_axis_name="s")
    cp = pltpu.CompilerParams()
    if "needs_layout_passes" in pltpu.CompilerParams.__dataclass_fields__:
        cp = dataclasses.replace(cp, needs_layout_passes=False)

    n_tiles = 32
    rows_per_tile = BATCH // n_tiles
    n_chunks = rows_per_tile // ROWS_PER_STEP

    @functools.partial(
        pl.kernel,
        mesh=mesh,
        out_type=jax.ShapeDtypeStruct((BATCH, OUT_F), jnp.float32),
        scratch_types=[
            pltpu.VMEM((OUT_F // 128, 128), jnp.int32),
            pltpu.VMEM((1, OUT_F), jnp.int32),
            pltpu.VMEM((2, ROWS_PER_STEP, IN_F), jnp.float32),
            pltpu.VMEM((2, ROWS_PER_STEP, OUT_F), jnp.float32),
        ] + [pltpu.SemaphoreType.DMA] * 10,
        compiler_params=cp,
    )
    def k(i_hbm, x_hbm, o_hbm, i8_vmem, i_vmem, xb, ob, *sems):
        pltpu.async_copy(i_hbm, i8_vmem, sems[0]).wait()
        for rr in range(OUT_F // 128):
            for kk in range(128 // LANES):
                i_vmem[0, pl.ds(rr * 128 + kk * LANES, LANES)] = (
                    i8_vmem[rr, pl.ds(kk * LANES, LANES)])
        idx_ref = i_vmem.at[0]
        n_split = 4
        part = ROWS_PER_STEP // n_split
        sin = (sems[0:n_split], sems[n_split:2 * n_split])
        sout = (sems[2 * n_split], sems[2 * n_split + 1])

        wid = jax.lax.axis_index("s") * 2 + jax.lax.axis_index("c")
        base = wid * rows_per_tile

        def in_copies(chunk, buf):
            row0 = base + chunk * ROWS_PER_STEP
            return tuple(
                pltpu.make_async_copy(
                    x_hbm.at[pl.ds(row0 + p * part, part)],
                    xb.at[buf].at[pl.ds(p * part, part)], sin[buf][p])
                for p in range(n_split))

        def out_copy(chunk, buf):
            return pltpu.make_async_copy(
                ob.at[buf],
                o_hbm.at[pl.ds(base + chunk * ROWS_PER_STEP, ROWS_PER_STEP)],
                sout[buf])

        for c in in_copies(0, 0):
            c.start()

        @pl.loop(0, n_chunks, step=2)
        def _(g):
            for b in range(2):
                gi = g + b

                @pl.when(gi + 1 < n_chunks)
                def _():
                    for c in in_copies(gi + 1, 1 - b):
                        c.start()

                for c in in_copies(gi, b):
                    c.wait()

                @pl.when(gi >= 2)
                def _():
                    out_copy(gi - 2, b).wait()

                x_vmem = xb.at[b]
                o_vmem = ob.at[b]

                @plsc.parallel_loop(0, OUT_F // LANES, unroll=8)
                def _(j):
                    cols = idx_ref[pl.ds(j * LANES, LANES)]
                    for r in range(ROWS_PER_STEP):
                        rows = jnp.full((LANES,), r, jnp.int32)
                        vals = plsc.load_gather(x_vmem, [rows, cols])
                        o_vmem[r, pl.ds(j * LANES, LANES)] = vals

                out_copy(gi, b).start()

        out_copy(n_chunks - 2, 0).wait()
        out_copy(n_chunks - 1, 1).wait()

    return k(idx, x)


def kernel(x, raw_weight):
    idx = _argmax(raw_weight)
    return _gather_sc(x, idx)
